# baseline (speedup 1.0000x reference)
import functools

import jax
import jax.numpy as jnp
from jax import lax
from jax.experimental import pallas as pl
from jax.experimental.pallas import tpu as pltpu
from jax.experimental.pallas import tpu_sc as plsc

N = 10000
E = 320000
EPS = 1e-5

_NC = 2
_NS = 16
_CHUNK = 128
_RPT = 80
_R0 = 160
_R1 = 0
_EPAD = _NC * _NS * _RPT * _CHUNK
_W = 128
_ACC_ROWS = 10112
_ZROWS = _ACC_ROWS // _NS
_ORT = 624
_TAIL = N - _NS * _ORT


@functools.cache
def _make_segsum():
  mesh = plsc.VectorSubcoreMesh(core_axis_name="c", subcore_axis_name="s")
  out_type = jax.ShapeDtypeStruct((_NC, N, _W), jnp.float32)
  grp = 40
  ngrp = _RPT // grp
  scratch = [
      pltpu.VMEM((grp, _CHUNK), jnp.int32),
      pltpu.VMEM((grp, _CHUNK), jnp.int32),
      pltpu.VMEM((2, _CHUNK, _W), jnp.float32),
      pltpu.VMEM_SHARED((_ACC_ROWS, _W), jnp.float32),
      pltpu.SemaphoreType.DMA,
      pltpu.SemaphoreType.DMA,
  ]

  def body(p_hbm, src_hbm, dst_hbm, z_hbm, out_hbm, sidx, didx, rows, acc,
           gsem0, gsem1):
    c = lax.axis_index("c")
    s = lax.axis_index("s")
    rpt = _R0 + c * (_R1 - _R0)
    row0 = c * (_NS * _R0) + s * rpt
    pltpu.sync_copy(z_hbm, acc.at[pl.ds(s * _ZROWS, _ZROWS)])
    plsc.subcore_barrier()

    def group(g, carry):
      pltpu.sync_copy(src_hbm.at[pl.ds(row0 + g * grp, grp)], sidx)
      pltpu.sync_copy(dst_hbm.at[pl.ds(row0 + g * grp, grp)], didx)
      pltpu.async_copy(p_hbm.at[sidx.at[0]], rows.at[0], gsem0)

      def pair(j, cc):
        k0 = 2 * j
        pltpu.async_copy(p_hbm.at[sidx.at[k0 + 1]], rows.at[1], gsem1)
        pltpu.make_async_copy(p_hbm.at[sidx.at[0]], rows.at[0], gsem0).wait()
        pltpu.sync_copy(rows.at[0], acc.at[didx.at[k0]], add=True)

        @pl.when(j < grp // 2 - 1)
        def _refill():
          pltpu.async_copy(p_hbm.at[sidx.at[k0 + 2]], rows.at[0], gsem0)

        pltpu.make_async_copy(p_hbm.at[sidx.at[0]], rows.at[1], gsem1).wait()
        pltpu.sync_copy(rows.at[1], acc.at[didx.at[k0 + 1]], add=True)
        return cc

      lax.fori_loop(0, grp // 2, pair, 0)
      return carry

    lax.fori_loop(0, rpt // grp, group, 0)
    plsc.subcore_barrier()
    r0 = s * _ORT
    pltpu.sync_copy(acc.at[pl.ds(r0, _ORT)], out_hbm.at[c, pl.ds(r0, _ORT)])

    @pl.when(s == _NS - 1)
    def _tail_copy():
      t0 = _NS * _ORT
      pltpu.sync_copy(acc.at[pl.ds(t0, _TAIL)],
                      out_hbm.at[c, pl.ds(t0, _TAIL)])

  return pl.kernel(body, mesh=mesh, out_type=out_type, scratch_types=scratch)


def _bn_relu(z, g, b):
  mu = jnp.mean(z, axis=0, keepdims=True)
  zc = z - mu
  var = jnp.mean(zc * zc, axis=0, keepdims=True)
  return jnp.maximum(g * zc * lax.rsqrt(var + EPS) + b, 0.0)


def _pre0_body(x_ref, wl_ref, wr_ref, bl_ref, ones_ref, p_ref, q_ref):
  x = x_ref[...]
  p = jnp.dot(x, wl_ref[...], preferred_element_type=jnp.float32)
  p_ref[...] = jnp.concatenate([p, ones_ref[...]], axis=1)
  q_ref[...] = (jnp.dot(x, wr_ref[...], preferred_element_type=jnp.float32)
                + bl_ref[...])


def _pre0(x, wl, wr, bl):
  ones = jnp.concatenate(
      [jnp.ones((N, 16), jnp.float32), jnp.zeros((N, 48), jnp.float32)],
      axis=1)
  return pl.pallas_call(
      _pre0_body,
      out_shape=[jax.ShapeDtypeStruct((N, _W), jnp.float32),
                 jax.ShapeDtypeStruct((N, 64), jnp.float32)],
  )(x, wl, wr, bl.reshape(1, 64), ones)


def _mid1_body(part_ref, q_ref, g_ref, b_ref, h_ref, deg_ref):
  pp = part_ref[...]
  sfull = pp[0] + pp[1]
  deg = jnp.maximum(sfull[:, 64:65], 1.0)
  deg_ref[...] = deg
  z = sfull[:, :64] / deg + q_ref[...]
  h = _bn_relu(z, g_ref[...], b_ref[...])
  h_ref[...] = jnp.pad(h, ((0, 0), (0, 64)))


def _mid1(part, q, gamma, beta):
  return pl.pallas_call(
      _mid1_body,
      out_shape=[jax.ShapeDtypeStruct((N, _W), jnp.float32),
                 jax.ShapeDtypeStruct((N, 1), jnp.float32)],
  )(part, q, gamma.reshape(1, 64), beta.reshape(1, 64))


def _mid2_body(part_ref, haug_ref, deg_ref, wl_ref, wr_ref, bl_ref, g_ref,
               b_ref, h_ref):
  pp = part_ref[...]
  sfull = pp[0] + pp[1]
  agg = sfull[:, :64] / deg_ref[...]
  hprev = haug_ref[...][:, :64]
  z = (jnp.dot(agg, wl_ref[...], preferred_element_type=jnp.float32)
       + jnp.dot(hprev, wr_ref[...], preferred_element_type=jnp.float32)
       + bl_ref[...])
  h = _bn_relu(z, g_ref[...], b_ref[...])
  h_ref[...] = jnp.pad(h, ((0, 0), (0, 64)))


def _mid2(part, haug, deg, wl, wr, bl, gamma, beta):
  return pl.pallas_call(
      _mid2_body,
      out_shape=jax.ShapeDtypeStruct((N, _W), jnp.float32),
  )(part, haug, deg, wl, wr, bl.reshape(1, 64), gamma.reshape(1, 64),
    beta.reshape(1, 64))


def _fin_body(part_ref, haug_ref, deg_ref, wl_ref, wr_ref, bl_ref, g_ref,
              b_ref, w1_ref, b1_ref, w2_ref, b2_ref, o_ref):
  pp = part_ref[...]
  sfull = pp[0] + pp[1]
  agg = sfull[:, :64] / deg_ref[...]
  hprev = haug_ref[...][:, :64]
  z = (jnp.dot(agg, wl_ref[...], preferred_element_type=jnp.float32)
       + jnp.dot(hprev, wr_ref[...], preferred_element_type=jnp.float32)
       + bl_ref[...])
  h = _bn_relu(z, g_ref[...], b_ref[...])
  u = jnp.maximum(
      jnp.dot(h, w1_ref[...], preferred_element_type=jnp.float32)
      + b1_ref[...], 0.0)
  o = jnp.dot(u, w2_ref[...], preferred_element_type=jnp.float32) + b2_ref[...]
  o_ref[...] = jax.nn.sigmoid(o)


def _fin(part, haug, deg, wl, wr, bl, gamma, beta, w1, b1, w2, b2):
  return pl.pallas_call(
      _fin_body,
      out_shape=jax.ShapeDtypeStruct((N, 1), jnp.float32),
  )(part, haug, deg, wl, wr, bl.reshape(1, 32), gamma.reshape(1, 32),
    beta.reshape(1, 32), w1, b1.reshape(1, 32), w2, b2.reshape(1, 1))


def kernel(x, edge_index, params):
  src = edge_index[0]
  dst = edge_index[1]
  pad = _EPAD - E
  src2d = jnp.concatenate(
      [src, jnp.zeros((pad,), jnp.int32)]).reshape(-1, _CHUNK)
  dst2d = jnp.concatenate(
      [dst, jnp.full((pad,), N, jnp.int32)]).reshape(-1, _CHUNK)
  z128 = jnp.zeros((_ZROWS, _W), jnp.float32)
  segsum = _make_segsum()

  p_aug, q0 = _pre0(x, params["Wl0"], params["Wr0"], params["bl0"])
  part0 = segsum(p_aug, src2d, dst2d, z128)
  h1aug, deg = _mid1(part0, q0, params["gamma0"], params["beta0"])
  part1 = segsum(h1aug, src2d, dst2d, z128)
  h2aug = _mid2(part1, h1aug, deg, params["Wl1"], params["Wr1"],
                params["bl1"], params["gamma1"], params["beta1"])
  part2 = segsum(h2aug, src2d, dst2d, z128)
  return _fin(part2, h2aug, deg, params["Wl2"], params["Wr2"], params["bl2"],
              params["gamma2"], params["beta2"], params["W1"], params["b1"],
              params["W2"], params["b2"])


# speedup vs baseline: 1.1616x; 1.1616x over previous
import functools

import jax
import jax.numpy as jnp
from jax import lax
from jax.experimental import pallas as pl
from jax.experimental.pallas import tpu as pltpu
from jax.experimental.pallas import tpu_sc as plsc

N = 10000
E = 320000
EPS = 1e-5

_NC = 2
_NS = 16
_CHUNK = 128
_RPT = 80
_R0 = 112
_R1 = 48
_EPAD = _NC * _NS * _RPT * _CHUNK
_W = 128
_ACC_ROWS = 10112
_ZROWS = _ACC_ROWS // _NS
_ORT = 624
_TAIL = N - _NS * _ORT


@functools.cache
def _make_segsum():
  mesh = plsc.VectorSubcoreMesh(core_axis_name="c", subcore_axis_name="s")
  out_type = jax.ShapeDtypeStruct((_NC, N, _W), jnp.float32)
  grp = 16
  ngrp = _RPT // grp
  scratch = [
      pltpu.VMEM((grp, _CHUNK), jnp.int32),
      pltpu.VMEM((grp, _CHUNK), jnp.int32),
      pltpu.VMEM((2, _CHUNK, _W), jnp.float32),
      pltpu.VMEM_SHARED((_ACC_ROWS, _W), jnp.float32),
      pltpu.SemaphoreType.DMA,
      pltpu.SemaphoreType.DMA,
  ]

  def body(p_hbm, src_hbm, dst_hbm, z_hbm, out_hbm, sidx, didx, rows, acc,
           gsem0, gsem1):
    c = lax.axis_index("c")
    s = lax.axis_index("s")
    rpt = _R0 + c * (_R1 - _R0)
    row0 = c * (_NS * _R0) + s * rpt
    pltpu.sync_copy(z_hbm, acc.at[pl.ds(s * _ZROWS, _ZROWS)])
    plsc.subcore_barrier()

    def group(g, carry):
      pltpu.sync_copy(src_hbm.at[pl.ds(row0 + g * grp, grp)], sidx)
      pltpu.sync_copy(dst_hbm.at[pl.ds(row0 + g * grp, grp)], didx)
      pltpu.async_copy(p_hbm.at[sidx.at[0]], rows.at[0], gsem0)

      def pair(j, cc):
        k0 = 2 * j
        pltpu.async_copy(p_hbm.at[sidx.at[k0 + 1]], rows.at[1], gsem1)
        pltpu.make_async_copy(p_hbm.at[sidx.at[0]], rows.at[0], gsem0).wait()
        pltpu.sync_copy(rows.at[0], acc.at[didx.at[k0]], add=True)

        @pl.when(j < grp // 2 - 1)
        def _refill():
          pltpu.async_copy(p_hbm.at[sidx.at[k0 + 2]], rows.at[0], gsem0)

        pltpu.make_async_copy(p_hbm.at[sidx.at[0]], rows.at[1], gsem1).wait()
        pltpu.sync_copy(rows.at[1], acc.at[didx.at[k0 + 1]], add=True)
        return cc

      lax.fori_loop(0, grp // 2, pair, 0)
      return carry

    lax.fori_loop(0, rpt // grp, group, 0)
    plsc.subcore_barrier()
    r0 = s * _ORT
    pltpu.sync_copy(acc.at[pl.ds(r0, _ORT)], out_hbm.at[c, pl.ds(r0, _ORT)])

    @pl.when(s == _NS - 1)
    def _tail_copy():
      t0 = _NS * _ORT
      pltpu.sync_copy(acc.at[pl.ds(t0, _TAIL)],
                      out_hbm.at[c, pl.ds(t0, _TAIL)])

  return pl.kernel(body, mesh=mesh, out_type=out_type, scratch_types=scratch)


def _bn_relu(z, g, b):
  mu = jnp.mean(z, axis=0, keepdims=True)
  zc = z - mu
  var = jnp.mean(zc * zc, axis=0, keepdims=True)
  return jnp.maximum(g * zc * lax.rsqrt(var + EPS) + b, 0.0)


def _pre0_body(x_ref, wl_ref, wr_ref, bl_ref, ones_ref, p_ref, q_ref):
  x = x_ref[...]
  p = jnp.dot(x, wl_ref[...], preferred_element_type=jnp.float32)
  p_ref[...] = jnp.concatenate([p, ones_ref[...]], axis=1)
  q_ref[...] = (jnp.dot(x, wr_ref[...], preferred_element_type=jnp.float32)
                + bl_ref[...])


def _pre0(x, wl, wr, bl):
  ones = jnp.concatenate(
      [jnp.ones((N, 16), jnp.float32), jnp.zeros((N, 48), jnp.float32)],
      axis=1)
  return pl.pallas_call(
      _pre0_body,
      out_shape=[jax.ShapeDtypeStruct((N, _W), jnp.float32),
                 jax.ShapeDtypeStruct((N, 64), jnp.float32)],
  )(x, wl, wr, bl.reshape(1, 64), ones)


def _mid1_body(part_ref, q_ref, g_ref, b_ref, h_ref, deg_ref):
  pp = part_ref[...]
  sfull = pp[0] + pp[1]
  deg = jnp.maximum(sfull[:, 64:65], 1.0)
  deg_ref[...] = deg
  z = sfull[:, :64] / deg + q_ref[...]
  h = _bn_relu(z, g_ref[...], b_ref[...])
  h_ref[...] = jnp.pad(h, ((0, 0), (0, 64)))


def _mid1(part, q, gamma, beta):
  return pl.pallas_call(
      _mid1_body,
      out_shape=[jax.ShapeDtypeStruct((N, _W), jnp.float32),
                 jax.ShapeDtypeStruct((N, 1), jnp.float32)],
  )(part, q, gamma.reshape(1, 64), beta.reshape(1, 64))


def _mid2_body(part_ref, haug_ref, deg_ref, wl_ref, wr_ref, bl_ref, g_ref,
               b_ref, h_ref):
  pp = part_ref[...]
  sfull = pp[0] + pp[1]
  agg = sfull[:, :64] / deg_ref[...]
  hprev = haug_ref[...][:, :64]
  z = (jnp.dot(agg, wl_ref[...], preferred_element_type=jnp.float32)
       + jnp.dot(hprev, wr_ref[...], preferred_element_type=jnp.float32)
       + bl_ref[...])
  h = _bn_relu(z, g_ref[...], b_ref[...])
  h_ref[...] = jnp.pad(h, ((0, 0), (0, 64)))


def _mid2(part, haug, deg, wl, wr, bl, gamma, beta):
  return pl.pallas_call(
      _mid2_body,
      out_shape=jax.ShapeDtypeStruct((N, _W), jnp.float32),
  )(part, haug, deg, wl, wr, bl.reshape(1, 64), gamma.reshape(1, 64),
    beta.reshape(1, 64))


def _fin_body(part_ref, haug_ref, deg_ref, wl_ref, wr_ref, bl_ref, g_ref,
              b_ref, w1_ref, b1_ref, w2_ref, b2_ref, o_ref):
  pp = part_ref[...]
  sfull = pp[0] + pp[1]
  agg = sfull[:, :64] / deg_ref[...]
  hprev = haug_ref[...][:, :64]
  z = (jnp.dot(agg, wl_ref[...], preferred_element_type=jnp.float32)
       + jnp.dot(hprev, wr_ref[...], preferred_element_type=jnp.float32)
       + bl_ref[...])
  h = _bn_relu(z, g_ref[...], b_ref[...])
  u = jnp.maximum(
      jnp.dot(h, w1_ref[...], preferred_element_type=jnp.float32)
      + b1_ref[...], 0.0)
  o = jnp.dot(u, w2_ref[...], preferred_element_type=jnp.float32) + b2_ref[...]
  o_ref[...] = jax.nn.sigmoid(o)


def _fin(part, haug, deg, wl, wr, bl, gamma, beta, w1, b1, w2, b2):
  return pl.pallas_call(
      _fin_body,
      out_shape=jax.ShapeDtypeStruct((N, 1), jnp.float32),
  )(part, haug, deg, wl, wr, bl.reshape(1, 32), gamma.reshape(1, 32),
    beta.reshape(1, 32), w1, b1.reshape(1, 32), w2, b2.reshape(1, 1))


def kernel(x, edge_index, params):
  src = edge_index[0]
  dst = edge_index[1]
  pad = _EPAD - E
  src2d = jnp.concatenate(
      [src, jnp.zeros((pad,), jnp.int32)]).reshape(-1, _CHUNK)
  dst2d = jnp.concatenate(
      [dst, jnp.full((pad,), N, jnp.int32)]).reshape(-1, _CHUNK)
  z128 = jnp.zeros((_ZROWS, _W), jnp.float32)
  segsum = _make_segsum()

  p_aug, q0 = _pre0(x, params["Wl0"], params["Wr0"], params["bl0"])
  part0 = segsum(p_aug, src2d, dst2d, z128)
  h1aug, deg = _mid1(part0, q0, params["gamma0"], params["beta0"])
  part1 = segsum(h1aug, src2d, dst2d, z128)
  h2aug = _mid2(part1, h1aug, deg, params["Wl1"], params["Wr1"],
                params["bl1"], params["gamma1"], params["beta1"])
  part2 = segsum(h2aug, src2d, dst2d, z128)
  return _fin(part2, h2aug, deg, params["Wl2"], params["Wr2"], params["bl2"],
              params["gamma2"], params["beta2"], params["W1"], params["b1"],
              params["W2"], params["b2"])


# speedup vs baseline: 1.1835x; 1.0189x over previous
import functools

import jax
import jax.numpy as jnp
from jax import lax
from jax.experimental import pallas as pl
from jax.experimental.pallas import tpu as pltpu
from jax.experimental.pallas import tpu_sc as plsc

N = 10000
E = 320000
EPS = 1e-5

_NC = 2
_NS = 16
_CHUNK = 128
_RPT = 80
_R0 = 120
_R1 = 40
_EPAD = _NC * _NS * _RPT * _CHUNK
_W = 128
_ACC_ROWS = 10112
_ZROWS = _ACC_ROWS // _NS
_ORT = 624
_TAIL = N - _NS * _ORT


@functools.cache
def _make_segsum():
  mesh = plsc.VectorSubcoreMesh(core_axis_name="c", subcore_axis_name="s")
  out_type = jax.ShapeDtypeStruct((_NC, N, _W), jnp.float32)
  grp = 40
  ngrp = _RPT // grp
  scratch = [
      pltpu.VMEM((grp, _CHUNK), jnp.int32),
      pltpu.VMEM((grp, _CHUNK), jnp.int32),
      pltpu.VMEM((2, _CHUNK, _W), jnp.float32),
      pltpu.VMEM_SHARED((_ACC_ROWS, _W), jnp.float32),
      pltpu.SemaphoreType.DMA,
      pltpu.SemaphoreType.DMA,
  ]

  def body(p_hbm, src_hbm, dst_hbm, z_hbm, out_hbm, sidx, didx, rows, acc,
           gsem0, gsem1):
    c = lax.axis_index("c")
    s = lax.axis_index("s")
    rpt = _R0 + c * (_R1 - _R0)
    row0 = c * (_NS * _R0) + s * rpt
    pltpu.sync_copy(z_hbm, acc.at[pl.ds(s * _ZROWS, _ZROWS)])
    plsc.subcore_barrier()

    def group(g, carry):
      pltpu.sync_copy(src_hbm.at[pl.ds(row0 + g * grp, grp)], sidx)
      pltpu.sync_copy(dst_hbm.at[pl.ds(row0 + g * grp, grp)], didx)
      pltpu.async_copy(p_hbm.at[sidx.at[0]], rows.at[0], gsem0)

      def pair(j, cc):
        k0 = 2 * j
        pltpu.async_copy(p_hbm.at[sidx.at[k0 + 1]], rows.at[1], gsem1)
        pltpu.make_async_copy(p_hbm.at[sidx.at[0]], rows.at[0], gsem0).wait()
        pltpu.sync_copy(rows.at[0], acc.at[didx.at[k0]], add=True)

        @pl.when(j < grp // 2 - 1)
        def _refill():
          pltpu.async_copy(p_hbm.at[sidx.at[k0 + 2]], rows.at[0], gsem0)

        pltpu.make_async_copy(p_hbm.at[sidx.at[0]], rows.at[1], gsem1).wait()
        pltpu.sync_copy(rows.at[1], acc.at[didx.at[k0 + 1]], add=True)
        return cc

      lax.fori_loop(0, grp // 2, pair, 0)
      return carry

    lax.fori_loop(0, rpt // grp, group, 0)
    plsc.subcore_barrier()
    r0 = s * _ORT
    pltpu.sync_copy(acc.at[pl.ds(r0, _ORT)], out_hbm.at[c, pl.ds(r0, _ORT)])

    @pl.when(s == _NS - 1)
    def _tail_copy():
      t0 = _NS * _ORT
      pltpu.sync_copy(acc.at[pl.ds(t0, _TAIL)],
                      out_hbm.at[c, pl.ds(t0, _TAIL)])

  return pl.kernel(body, mesh=mesh, out_type=out_type, scratch_types=scratch)


def _bn_relu(z, g, b):
  mu = jnp.mean(z, axis=0, keepdims=True)
  zc = z - mu
  var = jnp.mean(zc * zc, axis=0, keepdims=True)
  return jnp.maximum(g * zc * lax.rsqrt(var + EPS) + b, 0.0)


def _pre0_body(x_ref, wl_ref, wr_ref, bl_ref, ones_ref, p_ref, q_ref):
  x = x_ref[...]
  p = jnp.dot(x, wl_ref[...], preferred_element_type=jnp.float32)
  p_ref[...] = jnp.concatenate([p, ones_ref[...]], axis=1)
  q_ref[...] = (jnp.dot(x, wr_ref[...], preferred_element_type=jnp.float32)
                + bl_ref[...])


def _pre0(x, wl, wr, bl):
  ones = jnp.concatenate(
      [jnp.ones((N, 16), jnp.float32), jnp.zeros((N, 48), jnp.float32)],
      axis=1)
  return pl.pallas_call(
      _pre0_body,
      out_shape=[jax.ShapeDtypeStruct((N, _W), jnp.float32),
                 jax.ShapeDtypeStruct((N, 64), jnp.float32)],
  )(x, wl, wr, bl.reshape(1, 64), ones)


def _mid1_body(part_ref, q_ref, g_ref, b_ref, h_ref, deg_ref):
  pp = part_ref[...]
  sfull = pp[0] + pp[1]
  deg = jnp.maximum(sfull[:, 64:65], 1.0)
  deg_ref[...] = deg
  z = sfull[:, :64] / deg + q_ref[...]
  h = _bn_relu(z, g_ref[...], b_ref[...])
  h_ref[...] = jnp.pad(h, ((0, 0), (0, 64)))


def _mid1(part, q, gamma, beta):
  return pl.pallas_call(
      _mid1_body,
      out_shape=[jax.ShapeDtypeStruct((N, _W), jnp.float32),
                 jax.ShapeDtypeStruct((N, 1), jnp.float32)],
  )(part, q, gamma.reshape(1, 64), beta.reshape(1, 64))


def _mid2_body(part_ref, haug_ref, deg_ref, wl_ref, wr_ref, bl_ref, g_ref,
               b_ref, h_ref):
  pp = part_ref[...]
  sfull = pp[0] + pp[1]
  agg = sfull[:, :64] / deg_ref[...]
  hprev = haug_ref[...][:, :64]
  z = (jnp.dot(agg, wl_ref[...], preferred_element_type=jnp.float32)
       + jnp.dot(hprev, wr_ref[...], preferred_element_type=jnp.float32)
       + bl_ref[...])
  h = _bn_relu(z, g_ref[...], b_ref[...])
  h_ref[...] = jnp.pad(h, ((0, 0), (0, 64)))


def _mid2(part, haug, deg, wl, wr, bl, gamma, beta):
  return pl.pallas_call(
      _mid2_body,
      out_shape=jax.ShapeDtypeStruct((N, _W), jnp.float32),
  )(part, haug, deg, wl, wr, bl.reshape(1, 64), gamma.reshape(1, 64),
    beta.reshape(1, 64))


def _fin_body(part_ref, haug_ref, deg_ref, wl_ref, wr_ref, bl_ref, g_ref,
              b_ref, w1_ref, b1_ref, w2_ref, b2_ref, o_ref):
  pp = part_ref[...]
  sfull = pp[0] + pp[1]
  agg = sfull[:, :64] / deg_ref[...]
  hprev = haug_ref[...][:, :64]
  z = (jnp.dot(agg, wl_ref[...], preferred_element_type=jnp.float32)
       + jnp.dot(hprev, wr_ref[...], preferred_element_type=jnp.float32)
       + bl_ref[...])
  h = _bn_relu(z, g_ref[...], b_ref[...])
  u = jnp.maximum(
      jnp.dot(h, w1_ref[...], preferred_element_type=jnp.float32)
      + b1_ref[...], 0.0)
  o = jnp.dot(u, w2_ref[...], preferred_element_type=jnp.float32) + b2_ref[...]
  o_ref[...] = jax.nn.sigmoid(o)


def _fin(part, haug, deg, wl, wr, bl, gamma, beta, w1, b1, w2, b2):
  return pl.pallas_call(
      _fin_body,
      out_shape=jax.ShapeDtypeStruct((N, 1), jnp.float32),
  )(part, haug, deg, wl, wr, bl.reshape(1, 32), gamma.reshape(1, 32),
    beta.reshape(1, 32), w1, b1.reshape(1, 32), w2, b2.reshape(1, 1))


def kernel(x, edge_index, params):
  src = edge_index[0]
  dst = edge_index[1]
  pad = _EPAD - E
  src2d = jnp.concatenate(
      [src, jnp.zeros((pad,), jnp.int32)]).reshape(-1, _CHUNK)
  dst2d = jnp.concatenate(
      [dst, jnp.full((pad,), N, jnp.int32)]).reshape(-1, _CHUNK)
  z128 = jnp.zeros((_ZROWS, _W), jnp.float32)
  segsum = _make_segsum()

  p_aug, q0 = _pre0(x, params["Wl0"], params["Wr0"], params["bl0"])
  part0 = segsum(p_aug, src2d, dst2d, z128)
  h1aug, deg = _mid1(part0, q0, params["gamma0"], params["beta0"])
  part1 = segsum(h1aug, src2d, dst2d, z128)
  h2aug = _mid2(part1, h1aug, deg, params["Wl1"], params["Wr1"],
                params["bl1"], params["gamma1"], params["beta1"])
  part2 = segsum(h2aug, src2d, dst2d, z128)
  return _fin(part2, h2aug, deg, params["Wl2"], params["Wr2"], params["bl2"],
              params["gamma2"], params["beta2"], params["W1"], params["b1"],
              params["W2"], params["b2"])
